# baseline (speedup 1.0000x reference)
import jax
import jax.numpy as jnp
from jax import lax
from jax.experimental import pallas as pl
from jax.experimental.pallas import tpu as pltpu
from jax.experimental.pallas import tpu_sc as plsc

N = 10000
E = 320000
H = 128
T = 8

NC = 2
NS = 16
NW = NC * NS
EW = E // NW
C = 80
NCHUNK = EW // C
RPT = 624
TAIL = N - RPT * NS
TAIL_OFF = RPT * NS


NCH_ALL = NW * NCHUNK


def _prep_body(h_ref, w_ref, b_ref, src_ref, typ_ref, y_ref, gidx_ref):
    t = pl.program_id(1)
    y = lax.dot_general(h_ref[...], w_ref[0],
                        (((1,), (1,)), ((), ())),
                        preferred_element_type=jnp.float32)
    y_ref[...] = y + b_ref[0]

    @pl.when((t == 0) & (pl.program_id(0) == 0))
    def _aux():
        gidx_ref[...] = typ_ref[...] * N + src_ref[...]


def _prep(h, W_msg, b_msg, src, typ):
    BN = 2000
    nb = N // BN
    return pl.pallas_call(
        _prep_body,
        grid=(nb, T),
        in_specs=[
            pl.BlockSpec((BN, H), lambda i, t: (i, 0)),
            pl.BlockSpec((1, H, H), lambda i, t: (t, 0, 0)),
            pl.BlockSpec((1, 1, H), lambda i, t: (t, 0, 0)),
            pl.BlockSpec((E // 128, 128), lambda i, t: (0, 0)),
            pl.BlockSpec((E // 128, 128), lambda i, t: (0, 0)),
        ],
        out_specs=[
            pl.BlockSpec((BN, H), lambda i, t: (t * nb + i, 0)),
            pl.BlockSpec((E // 128, 128), lambda i, t: (0, 0)),
        ],
        out_shape=[
            jax.ShapeDtypeStruct((T * N, H), jnp.float32),
            jax.ShapeDtypeStruct((E // 128, 128), jnp.int32),
        ],
    )(h, W_msg, b_msg.reshape(T, 1, H), src.reshape(E // 128, 128),
      typ.reshape(E // 128, 128))


R = 4
K = 1
NPIPE = (NCHUNK // R) * R


def _sc_body(y_hbm, gidx_hbm, dst_hbm, out_hbm,
             gbuf, dbuf, rows_v, acc_sh, esem, gsem, ssem):
    cid = lax.axis_index("c")
    sid = lax.axis_index("s")
    wid = cid * NS + sid

    @pl.loop(0, C)
    def _zrow(rr):
        for i in range(H // 16):
            rows_v[0][rr, pl.ds(i * 16, 16)] = jnp.zeros((16,), jnp.float32)

    for k in range(RPT // C):
        pltpu.sync_copy(rows_v[0], acc_sh.at[pl.ds(sid * RPT + k * C, C)])
    rem = RPT % C
    pltpu.sync_copy(rows_v[0].at[pl.ds(0, rem)],
                    acc_sh.at[pl.ds(sid * RPT + (RPT // C) * C, rem)])

    @pl.when(sid == 0)
    def _zero_tail():
        pltpu.sync_copy(rows_v[0].at[pl.ds(0, TAIL)],
                        acc_sh.at[pl.ds(TAIL_OFF, TAIL)])

    plsc.subcore_barrier()

    def start_idx(c, b):
        base = wid * EW + c * C
        pltpu.async_copy(gidx_hbm.at[pl.ds(base, C)], gbuf[b], esem[b])
        pltpu.async_copy(dst_hbm.at[pl.ds(base, C)], dbuf[b], esem[b])

    def wait_idx(c, b):
        base = wid * EW + c * C
        pltpu.make_async_copy(gidx_hbm.at[pl.ds(base, C)], gbuf[b],
                              esem[b]).wait()
        pltpu.make_async_copy(dst_hbm.at[pl.ds(base, C)], dbuf[b],
                              esem[b]).wait()

    def start_gather(c, b):
        pltpu.async_copy(y_hbm.at[gbuf[b]], rows_v[b], gsem[b])

    def wait_gather(b):
        pltpu.make_async_copy(y_hbm.at[gbuf[b]], rows_v[b],
                              gsem[b]).wait()

    def start_scatter(b):
        pltpu.async_copy(rows_v[b], acc_sh.at[dbuf[b]], ssem[b],
                         add=True)

    def wait_scatter(b):
        pltpu.make_async_copy(rows_v[b], acc_sh.at[dbuf[b]],
                              ssem[b]).wait()

    for c in range(K):
        base = wid * EW + c * C
        pltpu.sync_copy(gidx_hbm.at[pl.ds(base, C)], gbuf[c])
        pltpu.sync_copy(dst_hbm.at[pl.ds(base, C)], dbuf[c])
        start_gather(c, c)
    start_idx(K, K)

    @pl.loop(0, NPIPE // R)
    def _grp(g):
        for r in range(R):
            c = g * R + r
            wait_gather(r)
            start_scatter(r)

            ci = c + K + 1
            bi = (r + K + 1) % R

            @pl.when(ci < NCHUNK)
            def _idx_prefetch():
                @pl.when(ci >= R)
                def _reclaim():
                    wait_scatter(bi)
                start_idx(ci, bi)

            cp = c + K
            bp = (r + K) % R

            @pl.when(cp < NCHUNK)
            def _gather_prefetch():
                wait_idx(cp, bp)
                start_gather(cp, bp)

    for c in range(NPIPE, NCHUNK):
        wait_gather(c % R)
        start_scatter(c % R)

    for b in range(R):
        wait_scatter(b)

    plsc.subcore_barrier()
    pltpu.sync_copy(acc_sh.at[pl.ds(sid * RPT, RPT)],
                    out_hbm.at[cid, pl.ds(sid * RPT, RPT)])

    @pl.when(sid == 0)
    def _write_tail():
        pltpu.sync_copy(acc_sh.at[pl.ds(TAIL_OFF, TAIL)],
                        out_hbm.at[cid, pl.ds(TAIL_OFF, TAIL)])


def _sc_scatter(y, gidx, dst):
    mesh = plsc.VectorSubcoreMesh(core_axis_name="c", subcore_axis_name="s",
                                  num_cores=NC, num_subcores=NS)
    f = pl.kernel(
        _sc_body,
        out_type=jax.ShapeDtypeStruct((NC, N, H), jnp.float32),
        mesh=mesh,
        scratch_types=[
            [pltpu.VMEM((C,), jnp.int32) for _ in range(R)],
            [pltpu.VMEM((C,), jnp.int32) for _ in range(R)],
            [pltpu.VMEM((C, H), jnp.float32) for _ in range(R)],
            pltpu.VMEM_SHARED((N, H), jnp.float32),
            [pltpu.SemaphoreType.DMA for _ in range(R)],
            [pltpu.SemaphoreType.DMA for _ in range(R)],
            [pltpu.SemaphoreType.DMA for _ in range(R)],
        ],
    )
    return f(y, gidx, dst)


def _gru_body(p_ref, h_ref, wih_ref, whh_ref, bih_ref, bhh_ref, o_ref):
    m = p_ref[0] + p_ref[1]
    hv = h_ref[...]
    gi = lax.dot_general(m, wih_ref[...], (((1,), (1,)), ((), ())),
                         preferred_element_type=jnp.float32) + bih_ref[...]
    gh = lax.dot_general(hv, whh_ref[...], (((1,), (1,)), ((), ())),
                         preferred_element_type=jnp.float32) + bhh_ref[...]
    i_r, i_z, i_n = gi[:, :H], gi[:, H:2 * H], gi[:, 2 * H:]
    h_r, h_z, h_n = gh[:, :H], gh[:, H:2 * H], gh[:, 2 * H:]
    r = jax.nn.sigmoid(i_r + h_r)
    z = jax.nn.sigmoid(i_z + h_z)
    n = jnp.tanh(i_n + r * h_n)
    o_ref[...] = (1.0 - z) * n + z * hv


def _gru(partials, h, wih, whh, bih, bhh):
    BN = 2000
    nb = N // BN
    return pl.pallas_call(
        _gru_body,
        grid=(nb,),
        in_specs=[
            pl.BlockSpec((NC, BN, H), lambda i: (0, i, 0)),
            pl.BlockSpec((BN, H), lambda i: (i, 0)),
            pl.BlockSpec((3 * H, H), lambda i: (0, 0)),
            pl.BlockSpec((3 * H, H), lambda i: (0, 0)),
            pl.BlockSpec((3 * H,), lambda i: (0,)),
            pl.BlockSpec((3 * H,), lambda i: (0,)),
        ],
        out_specs=pl.BlockSpec((BN, H), lambda i: (i, 0)),
        out_shape=jax.ShapeDtypeStruct((N, H), jnp.float32),
    )(partials, h, wih, whh, bih, bhh)


@jax.jit
def kernel(h, edge_index, edge_type, W_msg, b_msg, weight_ih, weight_hh,
           bias_ih, bias_hh):
    src = edge_index[0]
    dst = edge_index[1]
    y, gidx = _prep(h, W_msg, b_msg, src, edge_type)
    partials = _sc_scatter(y, gidx.reshape(E), dst)
    return _gru(partials, h, weight_ih, weight_hh, bias_ih, bias_hh)


# speedup vs baseline: 1.3286x; 1.3286x over previous
import jax
import jax.numpy as jnp
from jax import lax
from jax.experimental import pallas as pl
from jax.experimental.pallas import tpu as pltpu
from jax.experimental.pallas import tpu_sc as plsc

N = 10000
E = 320000
H = 128
T = 8

NC = 2
NS = 16
NW = NC * NS
EW = E // NW
C = 80
NCHUNK = EW // C
RPT = 624
TAIL = N - RPT * NS
TAIL_OFF = RPT * NS


NCH_ALL = NW * NCHUNK


def _prep_body(h_ref, w_ref, b_ref, src_ref, typ_ref, y_ref, gidx_ref):
    t = pl.program_id(1)
    y = lax.dot_general(h_ref[...], w_ref[0],
                        (((1,), (1,)), ((), ())),
                        preferred_element_type=jnp.float32)
    y_ref[...] = y + b_ref[0]

    @pl.when((t == 0) & (pl.program_id(0) == 0))
    def _aux():
        gidx_ref[...] = typ_ref[...] * N + src_ref[...]


def _prep(h, W_msg, b_msg, src, typ):
    BN = 2000
    nb = N // BN
    return pl.pallas_call(
        _prep_body,
        grid=(nb, T),
        in_specs=[
            pl.BlockSpec((BN, H), lambda i, t: (i, 0)),
            pl.BlockSpec((1, H, H), lambda i, t: (t, 0, 0)),
            pl.BlockSpec((1, 1, H), lambda i, t: (t, 0, 0)),
            pl.BlockSpec((E // 128, 128), lambda i, t: (0, 0)),
            pl.BlockSpec((E // 128, 128), lambda i, t: (0, 0)),
        ],
        out_specs=[
            pl.BlockSpec((BN, H), lambda i, t: (t * nb + i, 0)),
            pl.BlockSpec((E // 128, 128), lambda i, t: (0, 0)),
        ],
        out_shape=[
            jax.ShapeDtypeStruct((T * N, H), jnp.float32),
            jax.ShapeDtypeStruct((E // 128, 128), jnp.int32),
        ],
    )(h, W_msg, b_msg.reshape(T, 1, H), src.reshape(E // 128, 128),
      typ.reshape(E // 128, 128))


R = 4
K = 2
NPIPE = (NCHUNK // R) * R


def _sc_body(y_hbm, gidx_hbm, dst_hbm, out_hbm,
             gbuf, dbuf, rows_v, acc_sh, esem, gsem, ssem):
    cid = lax.axis_index("c")
    sid = lax.axis_index("s")
    wid = cid * NS + sid

    def start_idx(c, b):
        base = wid * EW + c * C
        pltpu.async_copy(gidx_hbm.at[pl.ds(base, C)], gbuf[b], esem[b])
        pltpu.async_copy(dst_hbm.at[pl.ds(base, C)], dbuf[b], esem[b])

    def wait_idx(c, b):
        base = wid * EW + c * C
        pltpu.make_async_copy(gidx_hbm.at[pl.ds(base, C)], gbuf[b],
                              esem[b]).wait()
        pltpu.make_async_copy(dst_hbm.at[pl.ds(base, C)], dbuf[b],
                              esem[b]).wait()

    def start_gather(c, b):
        pltpu.async_copy(y_hbm.at[gbuf[b]], rows_v[b], gsem[b])

    def wait_gather(b):
        pltpu.make_async_copy(y_hbm.at[gbuf[b]], rows_v[b],
                              gsem[b]).wait()

    def start_scatter(b):
        pltpu.async_copy(rows_v[b], acc_sh.at[dbuf[b]], ssem[b],
                         add=True)

    def wait_scatter(b):
        pltpu.make_async_copy(rows_v[b], acc_sh.at[dbuf[b]],
                              ssem[b]).wait()

    for c in range(K):
        base = wid * EW + c * C
        pltpu.sync_copy(gidx_hbm.at[pl.ds(base, C)], gbuf[c])
        pltpu.sync_copy(dst_hbm.at[pl.ds(base, C)], dbuf[c])
        start_gather(c, c)
    start_idx(K, K)

    zbuf = rows_v[R - 1]

    @pl.loop(0, C)
    def _zrow(rr):
        for i in range(H // 16):
            zbuf[rr, pl.ds(i * 16, 16)] = jnp.zeros((16,), jnp.float32)

    for k in range(RPT // C):
        pltpu.sync_copy(zbuf, acc_sh.at[pl.ds(sid * RPT + k * C, C)])
    rem = RPT % C
    pltpu.sync_copy(zbuf.at[pl.ds(0, rem)],
                    acc_sh.at[pl.ds(sid * RPT + (RPT // C) * C, rem)])

    @pl.when(sid == 0)
    def _zero_tail():
        pltpu.sync_copy(zbuf.at[pl.ds(0, TAIL)],
                        acc_sh.at[pl.ds(TAIL_OFF, TAIL)])

    plsc.subcore_barrier()

    @pl.loop(0, NPIPE // R)
    def _grp(g):
        for r in range(R):
            c = g * R + r
            wait_gather(r)
            start_scatter(r)

            ci = c + K + 1
            bi = (r + K + 1) % R

            @pl.when(ci < NCHUNK)
            def _idx_prefetch():
                @pl.when(ci >= R)
                def _reclaim():
                    wait_scatter(bi)
                start_idx(ci, bi)

            cp = c + K
            bp = (r + K) % R

            @pl.when(cp < NCHUNK)
            def _gather_prefetch():
                wait_idx(cp, bp)
                start_gather(cp, bp)

    for c in range(NPIPE, NCHUNK):
        wait_gather(c % R)
        start_scatter(c % R)

    for b in range(R):
        wait_scatter(b)

    plsc.subcore_barrier()
    pltpu.sync_copy(acc_sh.at[pl.ds(sid * RPT, RPT)],
                    out_hbm.at[cid, pl.ds(sid * RPT, RPT)])

    @pl.when(sid == 0)
    def _write_tail():
        pltpu.sync_copy(acc_sh.at[pl.ds(TAIL_OFF, TAIL)],
                        out_hbm.at[cid, pl.ds(TAIL_OFF, TAIL)])


def _sc_scatter(y, gidx, dst):
    mesh = plsc.VectorSubcoreMesh(core_axis_name="c", subcore_axis_name="s",
                                  num_cores=NC, num_subcores=NS)
    f = pl.kernel(
        _sc_body,
        out_type=jax.ShapeDtypeStruct((NC, N, H), jnp.float32),
        mesh=mesh,
        scratch_types=[
            [pltpu.VMEM((C,), jnp.int32) for _ in range(R)],
            [pltpu.VMEM((C,), jnp.int32) for _ in range(R)],
            [pltpu.VMEM((C, H), jnp.float32) for _ in range(R)],
            pltpu.VMEM_SHARED((N, H), jnp.float32),
            [pltpu.SemaphoreType.DMA for _ in range(R)],
            [pltpu.SemaphoreType.DMA for _ in range(R)],
            [pltpu.SemaphoreType.DMA for _ in range(R)],
        ],
    )
    return f(y, gidx, dst)


def _gru_body(p_ref, h_ref, wih_ref, whh_ref, bih_ref, bhh_ref, o_ref):
    m = p_ref[0] + p_ref[1]
    hv = h_ref[...]
    gi = lax.dot_general(m, wih_ref[...], (((1,), (1,)), ((), ())),
                         preferred_element_type=jnp.float32) + bih_ref[...]
    gh = lax.dot_general(hv, whh_ref[...], (((1,), (1,)), ((), ())),
                         preferred_element_type=jnp.float32) + bhh_ref[...]
    i_r, i_z, i_n = gi[:, :H], gi[:, H:2 * H], gi[:, 2 * H:]
    h_r, h_z, h_n = gh[:, :H], gh[:, H:2 * H], gh[:, 2 * H:]
    r = jax.nn.sigmoid(i_r + h_r)
    z = jax.nn.sigmoid(i_z + h_z)
    n = jnp.tanh(i_n + r * h_n)
    o_ref[...] = (1.0 - z) * n + z * hv


def _gru(partials, h, wih, whh, bih, bhh):
    BN = 2000
    nb = N // BN
    return pl.pallas_call(
        _gru_body,
        grid=(nb,),
        in_specs=[
            pl.BlockSpec((NC, BN, H), lambda i: (0, i, 0)),
            pl.BlockSpec((BN, H), lambda i: (i, 0)),
            pl.BlockSpec((3 * H, H), lambda i: (0, 0)),
            pl.BlockSpec((3 * H, H), lambda i: (0, 0)),
            pl.BlockSpec((3 * H,), lambda i: (0,)),
            pl.BlockSpec((3 * H,), lambda i: (0,)),
        ],
        out_specs=pl.BlockSpec((BN, H), lambda i: (i, 0)),
        out_shape=jax.ShapeDtypeStruct((N, H), jnp.float32),
    )(partials, h, wih, whh, bih, bhh)


@jax.jit
def kernel(h, edge_index, edge_type, W_msg, b_msg, weight_ih, weight_hh,
           bias_ih, bias_hh):
    src = edge_index[0]
    dst = edge_index[1]
    y, gidx = _prep(h, W_msg, b_msg, src, edge_type)
    partials = _sc_scatter(y, gidx.reshape(E), dst)
    return _gru(partials, h, weight_ih, weight_hh, bias_ih, bias_hh)
